# baseline (speedup 1.0000x reference)
import functools

import jax
import jax.numpy as jnp
from jax import lax
from jax.experimental import pallas as pl
from jax.experimental.pallas import tpu as pltpu
from jax.experimental.pallas import tpu_sc as plsc

DIM = 32
L = 16
NC = 2
NS = 16
NW = NC * NS

NRELS = 500
NRELP = 512
SLABR = 16
NSLAB = NRELP // SLABR

NTRIP = 250000
CHUNK = 7936
CHUNK_LAST = NTRIP - (NW - 1) * CHUNK
CHUNKP = CHUNK + NSLAB * (L - 1)
CHUNKPX = CHUNKP + L
STEPS = CHUNK // L
STEPS_LAST = CHUNK_LAST // L


def _lane_scalar(vec, ts):
    g = jnp.take(vec, jnp.full((L,), ts, jnp.int32), mode="fill")
    return g[0]


def _sc_body(trip_hbm, node_hbm, w_hbm, out_hbm,
             ids3,
             srt_src, srt_dst, srt_rel, srt_org,
             hist, out_v, srow2, drow2, wslab2, vred,
             cnt_sm, pos_sm,
             sem_w0, sem_w1, sem_rs0, sem_rd0, sem_rs1, sem_rd1):
    wid = lax.axis_index("s") * NC + lax.axis_index("c")
    base = wid * CHUNK
    is_last = wid == NW - 1
    steps_w = jnp.where(is_last, STEPS_LAST, STEPS)

    @pl.when(is_last)
    def _():
        pltpu.sync_copy(trip_hbm.at[pl.ds(base * 3, CHUNK_LAST * 3)],
                        ids3.at[pl.ds(0, CHUNK_LAST * 3)])

    @pl.when(jnp.logical_not(is_last))
    def _():
        pltpu.sync_copy(trip_hbm.at[pl.ds(base * 3, CHUNK * 3)], ids3)

    lane = lax.broadcasted_iota(jnp.int32, (L,), 0)
    ones_i = jnp.ones((L,), jnp.int32)
    zeros_i = jnp.zeros((L,), jnp.int32)

    def zero_body(i, c):
        hist[pl.ds(i * L, L)] = jnp.zeros((L,), jnp.int32)
        return c
    lax.fori_loop(0, NSLAB, zero_body, 0)

    def hist_body(k, c):
        rv = plsc.load_gather(ids3, [(k * L + lane) * 3 + 1])
        sid = lax.shift_right_logical(rv, 4)
        plsc.addupdate_scatter(hist, [sid * L + lane], ones_i)
        return c
    lax.fori_loop(0, steps_w, hist_body, 0)

    def b2_body(s, posb):
        hrow = hist[pl.ds(s * L, L)]
        cum = plsc.cumsum(hrow)
        cnt = cum[L - 1]
        hist[pl.ds(s * L, L)] = (cum - hrow) + posb
        cnt_sm[s] = cnt
        pos_sm[s] = posb
        return posb + ((cnt + (L - 1)) // L) * L
    lax.fori_loop(0, NSLAB, b2_body, 0)

    dumpv = jnp.full((L,), CHUNK, jnp.int32)
    zerov = jnp.zeros((L,), jnp.int32)

    def init_body(i, c):
        srt_src[pl.ds(i * L, L)] = zerov
        srt_dst[pl.ds(i * L, L)] = zerov
        srt_rel[pl.ds(i * L, L)] = zerov
        srt_org[pl.ds(i * L, L)] = dumpv
        return c
    lax.fori_loop(0, CHUNKPX // L, init_body, 0)

    def b3_body(k, c):
        off = k * L
        rowv = (off + lane) * 3
        sv = plsc.load_gather(ids3, [rowv])
        rv = plsc.load_gather(ids3, [rowv + 1])
        dv = plsc.load_gather(ids3, [rowv + 2])
        sid = lax.shift_right_logical(rv, 4)
        idx = sid * L + lane
        pos = plsc.load_gather(hist, [idx])
        plsc.store_scatter(srt_src, [pos], sv)
        plsc.store_scatter(srt_dst, [pos], dv)
        plsc.store_scatter(srt_rel, [pos], rv)
        plsc.store_scatter(srt_org, [pos], off + lane)
        plsc.addupdate_scatter(hist, [idx], ones_i)
        return c
    lax.fori_loop(0, steps_w, b3_body, 0)

    w_sems = (sem_w0, sem_w1)
    rs_sems = (sem_rs0, sem_rs1)
    rd_sems = (sem_rd0, sem_rd1)

    def _woff(s):
        return jnp.minimum(s * SLABR, NRELS - SLABR)

    def start_w(s, p):
        pltpu.async_copy(w_hbm.at[pl.ds(_woff(s), SLABR)], wslab2.at[p],
                         w_sems[p])

    def wait_w(p):
        pltpu.make_async_copy(w_hbm.at[pl.ds(0, SLABR)], wslab2.at[p],
                              w_sems[p]).wait()

    def start_rows(goff, p):
        sv = srt_src[pl.ds(goff, L)]
        dv = srt_dst[pl.ds(goff, L)]
        pltpu.async_copy(node_hbm.at[sv], srow2.at[p], rs_sems[p])
        pltpu.async_copy(node_hbm.at[dv], drow2.at[p], rd_sems[p])

    def wait_rows(p):
        pltpu.make_async_copy(node_hbm.at[pl.ds(0, L)], srow2.at[p],
                              rs_sems[p]).wait()
        pltpu.make_async_copy(node_hbm.at[pl.ds(0, L)], drow2.at[p],
                              rd_sems[p]).wait()

    def compute_group(s, posb, cnt, g, p, pw, roff):
        goff = posb + g * L
        rv = srt_rel[pl.ds(goff, L)]
        wslotv = jnp.minimum(jnp.maximum(rv - _woff(s), 0), SLABR - 1)
        orgv = srt_org[pl.ds(goff, L)]
        wslab = wslab2.at[pw]
        srows = srow2.at[p]
        drows = drow2.at[p]

        def trip_one(ts):
            wsl = _lane_scalar(wslotv, ts)
            s0 = srows[roff + ts, 0:L]
            s1 = srows[roff + ts, L:DIM]
            acc0 = jnp.zeros((L,), jnp.float32)
            acc1 = jnp.zeros((L,), jnp.float32)
            for i in range(L):
                acc0 = acc0 + s0[i] * wslab[wsl, i, 0:L]
                acc1 = acc1 + s0[i] * wslab[wsl, i, L:DIM]
            for i in range(L):
                acc0 = acc0 + s1[i] * wslab[wsl, L + i, 0:L]
                acc1 = acc1 + s1[i] * wslab[wsl, L + i, L:DIM]
            d0 = drows[roff + ts, 0:L]
            d1 = drows[roff + ts, L:DIM]
            vred[ts, 0:L] = acc0 * d0 + acc1 * d1

        def trip_body(tp, c3):
            trip_one(2 * tp)
            trip_one(2 * tp + 1)
            return c3
        lax.fori_loop(0, L // 2, trip_body, 0)

        score_vec = jnp.zeros((L,), jnp.float32)
        for j in range(L):
            col = plsc.load_gather(vred, [lane, jnp.full((L,), j, jnp.int32)])
            score_vec = score_vec + col
        maskv = (g * L + lane) < cnt
        plsc.store_scatter(out_v, [orgv], score_vec, mask=maskv)

    def do_slab(s, pw):
        cnt = cnt_sm[s]
        posb = pl.multiple_of(pos_sm[s], L)
        ngroups = (cnt + (L - 1)) // L
        wait_w(pw)

        @pl.when(ngroups > 0)
        def _():
            start_rows(posb, 0)

            def pair_body(gg, c2):
                g0 = 2 * gg

                @pl.when(g0 + 1 < ngroups)
                def _():
                    start_rows(posb + (g0 + 1) * L, 1)
                wait_rows(0)
                compute_group(s, posb, cnt, g0, 0, pw, 0)

                @pl.when(g0 + 2 < ngroups)
                def _():
                    start_rows(posb + (g0 + 2) * L, 0)

                @pl.when(g0 + 1 < ngroups)
                def _():
                    wait_rows(1)
                    compute_group(s, posb, cnt, g0 + 1, 1, pw, 0)
                return c2
            lax.fori_loop(0, (ngroups + 1) // 2, pair_body, 0)

    start_w(0, 0)

    def slab_pair(ss2, c):
        sa = 2 * ss2
        start_w(sa + 1, 1)
        do_slab(sa, 0)

        @pl.when(sa + 2 < NSLAB)
        def _():
            start_w(sa + 2, 0)
        do_slab(sa + 1, 1)
        return c
    lax.fori_loop(0, NSLAB // 2, slab_pair, 0)

    @pl.when(is_last)
    def _():
        pltpu.sync_copy(out_v.at[pl.ds(0, CHUNK_LAST)],
                        out_hbm.at[pl.ds(base, CHUNK_LAST)])

    @pl.when(jnp.logical_not(is_last))
    def _():
        pltpu.sync_copy(out_v.at[pl.ds(0, CHUNK)],
                        out_hbm.at[pl.ds(base, CHUNK)])


@functools.partial(jax.jit, static_argnames=("interpret",))
def _dist_mult_sc(trip, node_emb, w, interpret=False):
    mesh = plsc.VectorSubcoreMesh(core_axis_name="c", subcore_axis_name="s")
    kfn = pl.kernel(
        _sc_body,
        out_type=jax.ShapeDtypeStruct((NTRIP,), jnp.float32),
        mesh=mesh,
        compiler_params=pltpu.CompilerParams(needs_layout_passes=False,
                                             use_tc_tiling_on_sc=False),
        scratch_types=[
            pltpu.VMEM((CHUNK * 3,), jnp.int32),
            pltpu.VMEM((CHUNKPX,), jnp.int32),
            pltpu.VMEM((CHUNKPX,), jnp.int32),
            pltpu.VMEM((CHUNKPX,), jnp.int32),
            pltpu.VMEM((CHUNKPX,), jnp.int32),
            pltpu.VMEM((NSLAB * L,), jnp.int32),
            pltpu.VMEM((CHUNK + L,), jnp.float32),
            pltpu.VMEM((2, L, DIM), jnp.float32),
            pltpu.VMEM((2, L, DIM), jnp.float32),
            pltpu.VMEM((2, SLABR, DIM, DIM), jnp.float32),
            pltpu.VMEM((L, L), jnp.float32),
            pltpu.SMEM((NSLAB,), jnp.int32),
            pltpu.SMEM((NSLAB,), jnp.int32),
            pltpu.SemaphoreType.DMA,
            pltpu.SemaphoreType.DMA,
            pltpu.SemaphoreType.DMA,
            pltpu.SemaphoreType.DMA,
            pltpu.SemaphoreType.DMA,
            pltpu.SemaphoreType.DMA,
        ],
        interpret=interpret,
    )
    return kfn(trip, node_emb, w)


def kernel(triplets, node_emb, W):
    return _dist_mult_sc(triplets.astype(jnp.int32).reshape(-1), node_emb, W)


# speedup vs baseline: 1.2767x; 1.2767x over previous
import functools

import jax
import jax.numpy as jnp
from jax import lax
from jax.experimental import pallas as pl
from jax.experimental.pallas import tpu as pltpu
from jax.experimental.pallas import tpu_sc as plsc

DIM = 32
L = 16
NC = 2
NS = 16
NW = NC * NS

NRELS = 500
NRELP = 512
SLABR = 16
NSLAB = NRELP // SLABR

NTRIP = 250000
CHUNK = 7936
CHUNK_LAST = NTRIP - (NW - 1) * CHUNK
CHUNKP = CHUNK + NSLAB * (L - 1)
CHUNKPX = CHUNKP + L
STEPS = CHUNK // L
STEPS_LAST = CHUNK_LAST // L


def _lane_scalar(vec, ts):
    g = jnp.take(vec, jnp.full((L,), ts, jnp.int32), mode="fill")
    return g[0]


def _sc_body(trip_hbm, node_hbm, w_hbm, out_hbm,
             src_ids, rel_ids, dst_ids,
             srt_src, srt_dst, srt_rel, srt_org,
             hist, out_v, srow2, drow2, wslab2, vred,
             cnt_sm, pos_sm,
             sem_w0, sem_w1, sem_rs0, sem_rd0, sem_rs1, sem_rd1):
    wid = lax.axis_index("s") * NC + lax.axis_index("c")
    base = wid * CHUNK
    is_last = wid == NW - 1
    steps_w = jnp.where(is_last, STEPS_LAST, STEPS)

    @pl.when(is_last)
    def _():
        pltpu.sync_copy(trip_hbm.at[0, pl.ds(base, CHUNK_LAST)],
                        src_ids.at[pl.ds(0, CHUNK_LAST)])
        pltpu.sync_copy(trip_hbm.at[1, pl.ds(base, CHUNK_LAST)],
                        rel_ids.at[pl.ds(0, CHUNK_LAST)])
        pltpu.sync_copy(trip_hbm.at[2, pl.ds(base, CHUNK_LAST)],
                        dst_ids.at[pl.ds(0, CHUNK_LAST)])

    @pl.when(jnp.logical_not(is_last))
    def _():
        pltpu.sync_copy(trip_hbm.at[0, pl.ds(base, CHUNK)], src_ids)
        pltpu.sync_copy(trip_hbm.at[1, pl.ds(base, CHUNK)], rel_ids)
        pltpu.sync_copy(trip_hbm.at[2, pl.ds(base, CHUNK)], dst_ids)

    lane = lax.broadcasted_iota(jnp.int32, (L,), 0)
    ones_i = jnp.ones((L,), jnp.int32)
    zeros_i = jnp.zeros((L,), jnp.int32)

    def zero_body(i, c):
        hist[pl.ds(i * L, L)] = jnp.zeros((L,), jnp.int32)
        return c
    lax.fori_loop(0, NSLAB, zero_body, 0)

    def hist_body(k, c):
        rv = rel_ids[pl.ds(k * L, L)]
        sid = lax.shift_right_logical(rv, 4)
        plsc.addupdate_scatter(hist, [sid * L + lane], ones_i)
        return c
    lax.fori_loop(0, steps_w, hist_body, 0)

    def b2_body(s, posb):
        hrow = hist[pl.ds(s * L, L)]
        cum = plsc.cumsum(hrow)
        cnt = cum[L - 1]
        hist[pl.ds(s * L, L)] = (cum - hrow) + posb
        cnt_sm[s] = cnt
        pos_sm[s] = posb
        return posb + ((cnt + (L - 1)) // L) * L
    lax.fori_loop(0, NSLAB, b2_body, 0)

    dumpv = jnp.full((L,), CHUNK, jnp.int32)
    zerov = jnp.zeros((L,), jnp.int32)

    def init_body(i, c):
        srt_src[pl.ds(i * L, L)] = zerov
        srt_dst[pl.ds(i * L, L)] = zerov
        srt_rel[pl.ds(i * L, L)] = zerov
        srt_org[pl.ds(i * L, L)] = dumpv
        return c
    lax.fori_loop(0, CHUNKPX // L, init_body, 0)

    def b3_body(k, c):
        off = k * L
        rv = rel_ids[pl.ds(off, L)]
        sv = src_ids[pl.ds(off, L)]
        dv = dst_ids[pl.ds(off, L)]
        sid = lax.shift_right_logical(rv, 4)
        idx = sid * L + lane
        pos = plsc.load_gather(hist, [idx])
        plsc.store_scatter(srt_src, [pos], sv)
        plsc.store_scatter(srt_dst, [pos], dv)
        plsc.store_scatter(srt_rel, [pos], rv)
        plsc.store_scatter(srt_org, [pos], off + lane)
        plsc.addupdate_scatter(hist, [idx], ones_i)
        return c
    lax.fori_loop(0, steps_w, b3_body, 0)

    w_sems = (sem_w0, sem_w1)
    rs_sems = (sem_rs0, sem_rs1)
    rd_sems = (sem_rd0, sem_rd1)

    def _woff(s):
        return jnp.minimum(s * SLABR, NRELS - SLABR)

    def start_w(s, p):
        pltpu.async_copy(w_hbm.at[pl.ds(_woff(s), SLABR)], wslab2.at[p],
                         w_sems[p])

    def wait_w(p):
        pltpu.make_async_copy(w_hbm.at[pl.ds(0, SLABR)], wslab2.at[p],
                              w_sems[p]).wait()

    def start_rows(goff, p):
        sv = srt_src[pl.ds(goff, L)]
        dv = srt_dst[pl.ds(goff, L)]
        pltpu.async_copy(node_hbm.at[sv], srow2.at[p], rs_sems[p])
        pltpu.async_copy(node_hbm.at[dv], drow2.at[p], rd_sems[p])

    def wait_rows(p):
        pltpu.make_async_copy(node_hbm.at[pl.ds(0, L)], srow2.at[p],
                              rs_sems[p]).wait()
        pltpu.make_async_copy(node_hbm.at[pl.ds(0, L)], drow2.at[p],
                              rd_sems[p]).wait()

    def compute_group(s, posb, cnt, g, p, pw, roff):
        goff = posb + g * L
        rv = srt_rel[pl.ds(goff, L)]
        wslotv = jnp.minimum(jnp.maximum(rv - _woff(s), 0), SLABR - 1)
        orgv = srt_org[pl.ds(goff, L)]
        wslab = wslab2.at[pw]
        srows = srow2.at[p]
        drows = drow2.at[p]

        def trip_one(ts):
            wsl = _lane_scalar(wslotv, ts)
            s0 = srows[roff + ts, 0:L]
            s1 = srows[roff + ts, L:DIM]
            acc0 = jnp.zeros((L,), jnp.float32)
            acc1 = jnp.zeros((L,), jnp.float32)
            for i in range(L):
                acc0 = acc0 + s0[i] * wslab[wsl, i, 0:L]
                acc1 = acc1 + s0[i] * wslab[wsl, i, L:DIM]
            for i in range(L):
                acc0 = acc0 + s1[i] * wslab[wsl, L + i, 0:L]
                acc1 = acc1 + s1[i] * wslab[wsl, L + i, L:DIM]
            d0 = drows[roff + ts, 0:L]
            d1 = drows[roff + ts, L:DIM]
            vred[ts, 0:L] = acc0 * d0 + acc1 * d1

        def trip_body(tp, c3):
            trip_one(2 * tp)
            trip_one(2 * tp + 1)
            return c3
        lax.fori_loop(0, L // 2, trip_body, 0)

        score_vec = jnp.zeros((L,), jnp.float32)
        for j in range(L):
            col = plsc.load_gather(vred, [lane, jnp.full((L,), j, jnp.int32)])
            score_vec = score_vec + col
        maskv = (g * L + lane) < cnt
        plsc.store_scatter(out_v, [orgv], score_vec, mask=maskv)

    def do_slab(s, pw):
        cnt = cnt_sm[s]
        posb = pl.multiple_of(pos_sm[s], L)
        ngroups = (cnt + (L - 1)) // L
        wait_w(pw)

        @pl.when(ngroups > 0)
        def _():
            start_rows(posb, 0)

            def pair_body(gg, c2):
                g0 = 2 * gg

                @pl.when(g0 + 1 < ngroups)
                def _():
                    start_rows(posb + (g0 + 1) * L, 1)
                wait_rows(0)
                compute_group(s, posb, cnt, g0, 0, pw, 0)

                @pl.when(g0 + 2 < ngroups)
                def _():
                    start_rows(posb + (g0 + 2) * L, 0)

                @pl.when(g0 + 1 < ngroups)
                def _():
                    wait_rows(1)
                    compute_group(s, posb, cnt, g0 + 1, 1, pw, 0)
                return c2
            lax.fori_loop(0, (ngroups + 1) // 2, pair_body, 0)

    start_w(0, 0)

    def slab_pair(ss2, c):
        sa = 2 * ss2
        start_w(sa + 1, 1)
        do_slab(sa, 0)

        @pl.when(sa + 2 < NSLAB)
        def _():
            start_w(sa + 2, 0)
        do_slab(sa + 1, 1)
        return c
    lax.fori_loop(0, NSLAB // 2, slab_pair, 0)

    @pl.when(is_last)
    def _():
        pltpu.sync_copy(out_v.at[pl.ds(0, CHUNK_LAST)],
                        out_hbm.at[pl.ds(base, CHUNK_LAST)])

    @pl.when(jnp.logical_not(is_last))
    def _():
        pltpu.sync_copy(out_v.at[pl.ds(0, CHUNK)],
                        out_hbm.at[pl.ds(base, CHUNK)])


@functools.partial(jax.jit, static_argnames=("interpret",))
def _dist_mult_sc(trip, node_emb, w, interpret=False):
    mesh = plsc.VectorSubcoreMesh(core_axis_name="c", subcore_axis_name="s")
    kfn = pl.kernel(
        _sc_body,
        out_type=jax.ShapeDtypeStruct((NTRIP,), jnp.float32),
        mesh=mesh,
        compiler_params=pltpu.CompilerParams(needs_layout_passes=False,
                                             use_tc_tiling_on_sc=False),
        scratch_types=[
            pltpu.VMEM((CHUNK,), jnp.int32),
            pltpu.VMEM((CHUNK,), jnp.int32),
            pltpu.VMEM((CHUNK,), jnp.int32),
            pltpu.VMEM((CHUNKPX,), jnp.int32),
            pltpu.VMEM((CHUNKPX,), jnp.int32),
            pltpu.VMEM((CHUNKPX,), jnp.int32),
            pltpu.VMEM((CHUNKPX,), jnp.int32),
            pltpu.VMEM((NSLAB * L,), jnp.int32),
            pltpu.VMEM((CHUNK + L,), jnp.float32),
            pltpu.VMEM((2, L, DIM), jnp.float32),
            pltpu.VMEM((2, L, DIM), jnp.float32),
            pltpu.VMEM((2, SLABR, DIM, DIM), jnp.float32),
            pltpu.VMEM((L, L), jnp.float32),
            pltpu.SMEM((NSLAB,), jnp.int32),
            pltpu.SMEM((NSLAB,), jnp.int32),
            pltpu.SemaphoreType.DMA,
            pltpu.SemaphoreType.DMA,
            pltpu.SemaphoreType.DMA,
            pltpu.SemaphoreType.DMA,
            pltpu.SemaphoreType.DMA,
            pltpu.SemaphoreType.DMA,
        ],
        interpret=interpret,
    )
    return kfn(trip, node_emb, w)


def kernel(triplets, node_emb, W):
    return _dist_mult_sc(triplets.astype(jnp.int32).T, node_emb, W)


# speedup vs baseline: 1.2769x; 1.0002x over previous
import functools

import jax
import jax.numpy as jnp
from jax import lax
from jax.experimental import pallas as pl
from jax.experimental.pallas import tpu as pltpu
from jax.experimental.pallas import tpu_sc as plsc

DIM = 32
L = 16
NC = 2
NS = 16
NW = NC * NS

NRELS = 500
NRELP = 512
SLABR = 16
NSLAB = NRELP // SLABR

NTRIP = 250000
CHUNK = 7936
CHUNK_LAST = NTRIP - (NW - 1) * CHUNK
CHUNKP = CHUNK + NSLAB * (L - 1)
CHUNKPX = CHUNKP + L
STEPS = CHUNK // L
STEPS_LAST = CHUNK_LAST // L


def _lane_scalar(vec, ts):
    g = jnp.take(vec, jnp.full((L,), ts, jnp.int32), mode="fill")
    return g[0]


def _sc_body(trip_hbm, node_hbm, w_hbm, out_hbm,
             src_ids, rel_ids, dst_ids,
             srt_src, srt_dst, srt_rel, srt_org,
             hist, out_v, srow2, drow2, wslab2, vred,
             cnt_sm, pos_sm,
             sem_w0, sem_w1, sem_rs0, sem_rd0, sem_rs1, sem_rd1):
    wid = lax.axis_index("s") * NC + lax.axis_index("c")
    base = wid * CHUNK
    is_last = wid == NW - 1
    steps_w = jnp.where(is_last, STEPS_LAST, STEPS)

    @pl.when(is_last)
    def _():
        pltpu.sync_copy(trip_hbm.at[0, pl.ds(base, CHUNK_LAST)],
                        src_ids.at[pl.ds(0, CHUNK_LAST)])
        pltpu.sync_copy(trip_hbm.at[1, pl.ds(base, CHUNK_LAST)],
                        rel_ids.at[pl.ds(0, CHUNK_LAST)])
        pltpu.sync_copy(trip_hbm.at[2, pl.ds(base, CHUNK_LAST)],
                        dst_ids.at[pl.ds(0, CHUNK_LAST)])

    @pl.when(jnp.logical_not(is_last))
    def _():
        pltpu.sync_copy(trip_hbm.at[0, pl.ds(base, CHUNK)], src_ids)
        pltpu.sync_copy(trip_hbm.at[1, pl.ds(base, CHUNK)], rel_ids)
        pltpu.sync_copy(trip_hbm.at[2, pl.ds(base, CHUNK)], dst_ids)

    lane = lax.broadcasted_iota(jnp.int32, (L,), 0)
    ones_i = jnp.ones((L,), jnp.int32)

    def zero_body(i, c):
        hist[pl.ds(i * L, L)] = jnp.zeros((L,), jnp.int32)
        return c
    lax.fori_loop(0, NSLAB, zero_body, 0)

    def hist_body(k, c):
        rv = rel_ids[pl.ds(k * L, L)]
        sid = lax.shift_right_logical(rv, 4)
        plsc.addupdate_scatter(hist, [sid * L + lane], ones_i)
        return c
    lax.fori_loop(0, steps_w, hist_body, 0)

    def b2_body(s, posb):
        hrow = hist[pl.ds(s * L, L)]
        cum = plsc.cumsum(hrow)
        cnt = cum[L - 1]
        hist[pl.ds(s * L, L)] = (cum - hrow) + posb
        cnt_sm[s] = cnt
        pos_sm[s] = posb
        return posb + ((cnt + (L - 1)) // L) * L
    lax.fori_loop(0, NSLAB, b2_body, 0)

    dumpv = jnp.full((L,), CHUNK, jnp.int32)
    zerov = jnp.zeros((L,), jnp.int32)

    def init_body(i, c):
        srt_src[pl.ds(i * L, L)] = zerov
        srt_dst[pl.ds(i * L, L)] = zerov
        srt_rel[pl.ds(i * L, L)] = zerov
        srt_org[pl.ds(i * L, L)] = dumpv
        return c
    lax.fori_loop(0, CHUNKPX // L, init_body, 0)

    def b3_body(k, c):
        off = k * L
        rv = rel_ids[pl.ds(off, L)]
        sv = src_ids[pl.ds(off, L)]
        dv = dst_ids[pl.ds(off, L)]
        sid = lax.shift_right_logical(rv, 4)
        idx = sid * L + lane
        pos = plsc.load_gather(hist, [idx])
        plsc.store_scatter(srt_src, [pos], sv)
        plsc.store_scatter(srt_dst, [pos], dv)
        plsc.store_scatter(srt_rel, [pos], rv)
        plsc.store_scatter(srt_org, [pos], off + lane)
        plsc.addupdate_scatter(hist, [idx], ones_i)
        return c
    lax.fori_loop(0, steps_w, b3_body, 0)

    w_sems = (sem_w0, sem_w1)
    rs_sems = (sem_rs0, sem_rs1)
    rd_sems = (sem_rd0, sem_rd1)

    def _woff(s):
        return jnp.minimum(s * SLABR, NRELS - SLABR)

    def start_w(s, p):
        pltpu.async_copy(w_hbm.at[pl.ds(_woff(s), SLABR)], wslab2.at[p],
                         w_sems[p])

    def wait_w(p):
        pltpu.make_async_copy(w_hbm.at[pl.ds(0, SLABR)], wslab2.at[p],
                              w_sems[p]).wait()

    def start_rows(goff, p):
        sv = srt_src[pl.ds(goff, L)]
        dv = srt_dst[pl.ds(goff, L)]
        pltpu.async_copy(node_hbm.at[sv], srow2.at[p], rs_sems[p])
        pltpu.async_copy(node_hbm.at[dv], drow2.at[p], rd_sems[p])

    def wait_rows(p):
        pltpu.make_async_copy(node_hbm.at[pl.ds(0, L)], srow2.at[p],
                              rs_sems[p]).wait()
        pltpu.make_async_copy(node_hbm.at[pl.ds(0, L)], drow2.at[p],
                              rd_sems[p]).wait()

    def compute_group(s, posb, cnt, g, p, pw, roff):
        goff = posb + g * L
        rv = srt_rel[pl.ds(goff, L)]
        wslotv = jnp.minimum(jnp.maximum(rv - _woff(s), 0), SLABR - 1)
        orgv = srt_org[pl.ds(goff, L)]
        wslab = wslab2.at[pw]
        srows = srow2.at[p]
        drows = drow2.at[p]

        def trip_one(ts):
            wsl = _lane_scalar(wslotv, ts)
            s0 = srows[roff + ts, 0:L]
            s1 = srows[roff + ts, L:DIM]
            acc0 = jnp.zeros((L,), jnp.float32)
            acc1 = jnp.zeros((L,), jnp.float32)
            for i in range(L):
                acc0 = acc0 + s0[i] * wslab[wsl, i, 0:L]
                acc1 = acc1 + s0[i] * wslab[wsl, i, L:DIM]
            for i in range(L):
                acc0 = acc0 + s1[i] * wslab[wsl, L + i, 0:L]
                acc1 = acc1 + s1[i] * wslab[wsl, L + i, L:DIM]
            d0 = drows[roff + ts, 0:L]
            d1 = drows[roff + ts, L:DIM]
            vred[ts, 0:L] = acc0 * d0 + acc1 * d1

        def trip_body(tp, c3):
            trip_one(2 * tp)
            trip_one(2 * tp + 1)
            return c3
        lax.fori_loop(0, L // 2, trip_body, 0)

        score_vec = jnp.zeros((L,), jnp.float32)
        for j in range(L):
            col = plsc.load_gather(vred, [lane, jnp.full((L,), j, jnp.int32)])
            score_vec = score_vec + col
        maskv = (g * L + lane) < cnt
        plsc.store_scatter(out_v, [orgv], score_vec, mask=maskv)

    def do_slab(s, pw):
        cnt = cnt_sm[s]
        posb = pl.multiple_of(pos_sm[s], L)
        ngroups = (cnt + (L - 1)) // L
        wait_w(pw)

        @pl.when(ngroups > 0)
        def _():
            start_rows(posb, 0)

            def pair_body(gg, c2):
                g0 = 2 * gg

                @pl.when(g0 + 1 < ngroups)
                def _():
                    start_rows(posb + (g0 + 1) * L, 1)
                wait_rows(0)
                compute_group(s, posb, cnt, g0, 0, pw, 0)

                @pl.when(g0 + 2 < ngroups)
                def _():
                    start_rows(posb + (g0 + 2) * L, 0)

                @pl.when(g0 + 1 < ngroups)
                def _():
                    wait_rows(1)
                    compute_group(s, posb, cnt, g0 + 1, 1, pw, 0)
                return c2
            lax.fori_loop(0, (ngroups + 1) // 2, pair_body, 0)

    start_w(0, 0)

    def slab_pair(ss2, c):
        sa = 2 * ss2
        start_w(sa + 1, 1)
        do_slab(sa, 0)

        @pl.when(sa + 2 < NSLAB)
        def _():
            start_w(sa + 2, 0)
        do_slab(sa + 1, 1)
        return c
    lax.fori_loop(0, NSLAB // 2, slab_pair, 0)

    @pl.when(is_last)
    def _():
        pltpu.sync_copy(out_v.at[pl.ds(0, CHUNK_LAST)],
                        out_hbm.at[pl.ds(base, CHUNK_LAST)])

    @pl.when(jnp.logical_not(is_last))
    def _():
        pltpu.sync_copy(out_v.at[pl.ds(0, CHUNK)],
                        out_hbm.at[pl.ds(base, CHUNK)])


@functools.partial(jax.jit, static_argnames=("interpret",))
def _dist_mult_sc(trip, node_emb, w, interpret=False):
    mesh = plsc.VectorSubcoreMesh(core_axis_name="c", subcore_axis_name="s")
    kfn = pl.kernel(
        _sc_body,
        out_type=jax.ShapeDtypeStruct((NTRIP,), jnp.float32),
        mesh=mesh,
        compiler_params=pltpu.CompilerParams(needs_layout_passes=False,
                                             use_tc_tiling_on_sc=False),
        scratch_types=[
            pltpu.VMEM((CHUNK,), jnp.int32),
            pltpu.VMEM((CHUNK,), jnp.int32),
            pltpu.VMEM((CHUNK,), jnp.int32),
            pltpu.VMEM((CHUNKPX,), jnp.int32),
            pltpu.VMEM((CHUNKPX,), jnp.int32),
            pltpu.VMEM((CHUNKPX,), jnp.int32),
            pltpu.VMEM((CHUNKPX,), jnp.int32),
            pltpu.VMEM((NSLAB * L,), jnp.int32),
            pltpu.VMEM((CHUNK + L,), jnp.float32),
            pltpu.VMEM((2, L, DIM), jnp.float32),
            pltpu.VMEM((2, L, DIM), jnp.float32),
            pltpu.VMEM((2, SLABR, DIM, DIM), jnp.float32),
            pltpu.VMEM((L, L), jnp.float32),
            pltpu.SMEM((NSLAB,), jnp.int32),
            pltpu.SMEM((NSLAB,), jnp.int32),
            pltpu.SemaphoreType.DMA,
            pltpu.SemaphoreType.DMA,
            pltpu.SemaphoreType.DMA,
            pltpu.SemaphoreType.DMA,
            pltpu.SemaphoreType.DMA,
            pltpu.SemaphoreType.DMA,
        ],
        interpret=interpret,
    )
    return kfn(trip, node_emb, w)


def kernel(triplets, node_emb, W):
    return _dist_mult_sc(triplets.astype(jnp.int32).T, node_emb, W)


# speedup vs baseline: 1.4025x; 1.0983x over previous
import functools

import jax
import jax.numpy as jnp
from jax import lax
from jax.experimental import pallas as pl
from jax.experimental.pallas import tpu as pltpu
from jax.experimental.pallas import tpu_sc as plsc

DIM = 32
L = 16
NC = 2
NS = 16
NW = NC * NS

NRELS = 500
NRELP = 512
SLABR = 16
NSLAB = NRELP // SLABR

NTRIP = 250000
CHUNK = 7936
CHUNK_LAST = NTRIP - (NW - 1) * CHUNK
CHUNKP = CHUNK + NSLAB * (L - 1)
CHUNKPX = CHUNKP + L
STEPS = CHUNK // L
STEPS_LAST = CHUNK_LAST // L


def _lane_scalar(vec, ts):
    g = jnp.take(vec, jnp.full((L,), ts, jnp.int32), mode="fill")
    return g[0]


def _sc_body(trip_hbm, node_hbm, w_hbm, out_hbm,
             src_ids, rel_ids, dst_ids,
             srt_src, srt_dst, srt_rel, srt_org,
             hist, out_v, srow2, drow2, wslab2, vred,
             cnt_sm, pos_sm,
             sem_w0, sem_w1, sem_rs0, sem_rd0, sem_rs1, sem_rd1):
    wid = lax.axis_index("s") * NC + lax.axis_index("c")
    base = wid * CHUNK
    is_last = wid == NW - 1
    steps_w = jnp.where(is_last, STEPS_LAST, STEPS)

    @pl.when(is_last)
    def _():
        pltpu.sync_copy(trip_hbm.at[0, pl.ds(base, CHUNK_LAST)],
                        src_ids.at[pl.ds(0, CHUNK_LAST)])
        pltpu.sync_copy(trip_hbm.at[1, pl.ds(base, CHUNK_LAST)],
                        rel_ids.at[pl.ds(0, CHUNK_LAST)])
        pltpu.sync_copy(trip_hbm.at[2, pl.ds(base, CHUNK_LAST)],
                        dst_ids.at[pl.ds(0, CHUNK_LAST)])

    @pl.when(jnp.logical_not(is_last))
    def _():
        pltpu.sync_copy(trip_hbm.at[0, pl.ds(base, CHUNK)], src_ids)
        pltpu.sync_copy(trip_hbm.at[1, pl.ds(base, CHUNK)], rel_ids)
        pltpu.sync_copy(trip_hbm.at[2, pl.ds(base, CHUNK)], dst_ids)

    lane = lax.broadcasted_iota(jnp.int32, (L,), 0)
    ones_i = jnp.ones((L,), jnp.int32)

    def zero_body(i, c):
        hist[pl.ds(i * L, L)] = jnp.zeros((L,), jnp.int32)
        return c
    lax.fori_loop(0, NSLAB, zero_body, 0)

    def hist_body(k, c):
        rv = rel_ids[pl.ds(k * L, L)]
        sid = lax.shift_right_logical(rv, 4)
        plsc.addupdate_scatter(hist, [sid * L + lane], ones_i)
        return c
    lax.fori_loop(0, steps_w, hist_body, 0)

    def b2_body(s, posb):
        hrow = hist[pl.ds(s * L, L)]
        cum = plsc.cumsum(hrow)
        cnt = cum[L - 1]
        hist[pl.ds(s * L, L)] = (cum - hrow) + posb
        cnt_sm[s] = cnt
        pos_sm[s] = posb
        return posb + ((cnt + (L - 1)) // L) * L
    lax.fori_loop(0, NSLAB, b2_body, 0)

    dumpv = jnp.full((L,), CHUNK, jnp.int32)
    zerov = jnp.zeros((L,), jnp.int32)

    def init_body(i, c):
        srt_src[pl.ds(i * L, L)] = zerov
        srt_dst[pl.ds(i * L, L)] = zerov
        srt_rel[pl.ds(i * L, L)] = zerov
        srt_org[pl.ds(i * L, L)] = dumpv
        return c
    lax.fori_loop(0, CHUNKPX // L, init_body, 0)

    def b3_body(k, c):
        off = k * L
        rv = rel_ids[pl.ds(off, L)]
        sv = src_ids[pl.ds(off, L)]
        dv = dst_ids[pl.ds(off, L)]
        sid = lax.shift_right_logical(rv, 4)
        idx = sid * L + lane
        pos = plsc.load_gather(hist, [idx])
        plsc.store_scatter(srt_src, [pos], sv)
        plsc.store_scatter(srt_dst, [pos], dv)
        plsc.store_scatter(srt_rel, [pos], rv)
        plsc.store_scatter(srt_org, [pos], off + lane)
        plsc.addupdate_scatter(hist, [idx], ones_i)
        return c
    lax.fori_loop(0, steps_w, b3_body, 0)

    w_sems = (sem_w0, sem_w1)
    rs_sems = (sem_rs0, sem_rs1)
    rd_sems = (sem_rd0, sem_rd1)

    def _woff(s):
        return jnp.minimum(s * SLABR, NRELS - SLABR)

    def start_w(s, p):
        pltpu.async_copy(w_hbm.at[pl.ds(_woff(s), SLABR)], wslab2.at[p],
                         w_sems[p])

    def wait_w(p):
        pltpu.make_async_copy(w_hbm.at[pl.ds(0, SLABR)], wslab2.at[p],
                              w_sems[p]).wait()

    def start_rows(goff, p):
        sv = srt_src[pl.ds(goff, L)]
        dv = srt_dst[pl.ds(goff, L)]
        pltpu.async_copy(node_hbm.at[sv], srow2.at[p], rs_sems[p])
        pltpu.async_copy(node_hbm.at[dv], drow2.at[p], rd_sems[p])

    def wait_rows(p):
        pltpu.make_async_copy(node_hbm.at[pl.ds(0, L)], srow2.at[p],
                              rs_sems[p]).wait()
        pltpu.make_async_copy(node_hbm.at[pl.ds(0, L)], drow2.at[p],
                              rd_sems[p]).wait()

    def compute_group(s, posb, cnt, g, p, pw, roff):
        goff = posb + g * L
        rv = srt_rel[pl.ds(goff, L)]
        wslotv = jnp.minimum(jnp.maximum(rv - _woff(s), 0), SLABR - 1)
        orgv = srt_org[pl.ds(goff, L)]
        wslab = wslab2.at[pw]
        srows = srow2.at[p]
        drows = drow2.at[p]

        def trip_one(ts):
            wsl = _lane_scalar(wslotv, ts)
            s0 = srows[roff + ts, 0:L]
            s1 = srows[roff + ts, L:DIM]
            acc0 = jnp.zeros((L,), jnp.float32)
            acc1 = jnp.zeros((L,), jnp.float32)
            for i in range(L):
                acc0 = acc0 + s0[i] * wslab[wsl, i, 0:L]
                acc1 = acc1 + s0[i] * wslab[wsl, i, L:DIM]
            for i in range(L):
                acc0 = acc0 + s1[i] * wslab[wsl, L + i, 0:L]
                acc1 = acc1 + s1[i] * wslab[wsl, L + i, L:DIM]
            d0 = drows[roff + ts, 0:L]
            d1 = drows[roff + ts, L:DIM]
            vred[ts, 0:L] = acc0 * d0 + acc1 * d1

        @plsc.parallel_loop(0, L, 1, unroll=2)
        def _(ts):
            trip_one(ts)

        score_vec = jnp.zeros((L,), jnp.float32)
        for j in range(L):
            col = plsc.load_gather(vred, [lane, jnp.full((L,), j, jnp.int32)])
            score_vec = score_vec + col
        maskv = (g * L + lane) < cnt
        plsc.store_scatter(out_v, [orgv], score_vec, mask=maskv)

    def do_slab(s, pw):
        cnt = cnt_sm[s]
        posb = pl.multiple_of(pos_sm[s], L)
        ngroups = (cnt + (L - 1)) // L
        wait_w(pw)

        @pl.when(ngroups > 0)
        def _():
            start_rows(posb, 0)

            def pair_body(gg, c2):
                g0 = 2 * gg

                @pl.when(g0 + 1 < ngroups)
                def _():
                    start_rows(posb + (g0 + 1) * L, 1)
                wait_rows(0)
                compute_group(s, posb, cnt, g0, 0, pw, 0)

                @pl.when(g0 + 2 < ngroups)
                def _():
                    start_rows(posb + (g0 + 2) * L, 0)

                @pl.when(g0 + 1 < ngroups)
                def _():
                    wait_rows(1)
                    compute_group(s, posb, cnt, g0 + 1, 1, pw, 0)
                return c2
            lax.fori_loop(0, (ngroups + 1) // 2, pair_body, 0)

    start_w(0, 0)

    def slab_pair(ss2, c):
        sa = 2 * ss2
        start_w(sa + 1, 1)
        do_slab(sa, 0)

        @pl.when(sa + 2 < NSLAB)
        def _():
            start_w(sa + 2, 0)
        do_slab(sa + 1, 1)
        return c
    lax.fori_loop(0, NSLAB // 2, slab_pair, 0)

    @pl.when(is_last)
    def _():
        pltpu.sync_copy(out_v.at[pl.ds(0, CHUNK_LAST)],
                        out_hbm.at[pl.ds(base, CHUNK_LAST)])

    @pl.when(jnp.logical_not(is_last))
    def _():
        pltpu.sync_copy(out_v.at[pl.ds(0, CHUNK)],
                        out_hbm.at[pl.ds(base, CHUNK)])


@functools.partial(jax.jit, static_argnames=("interpret",))
def _dist_mult_sc(trip, node_emb, w, interpret=False):
    mesh = plsc.VectorSubcoreMesh(core_axis_name="c", subcore_axis_name="s")
    kfn = pl.kernel(
        _sc_body,
        out_type=jax.ShapeDtypeStruct((NTRIP,), jnp.float32),
        mesh=mesh,
        compiler_params=pltpu.CompilerParams(needs_layout_passes=False,
                                             use_tc_tiling_on_sc=False),
        scratch_types=[
            pltpu.VMEM((CHUNK,), jnp.int32),
            pltpu.VMEM((CHUNK,), jnp.int32),
            pltpu.VMEM((CHUNK,), jnp.int32),
            pltpu.VMEM((CHUNKPX,), jnp.int32),
            pltpu.VMEM((CHUNKPX,), jnp.int32),
            pltpu.VMEM((CHUNKPX,), jnp.int32),
            pltpu.VMEM((CHUNKPX,), jnp.int32),
            pltpu.VMEM((NSLAB * L,), jnp.int32),
            pltpu.VMEM((CHUNK + L,), jnp.float32),
            pltpu.VMEM((2, L, DIM), jnp.float32),
            pltpu.VMEM((2, L, DIM), jnp.float32),
            pltpu.VMEM((2, SLABR, DIM, DIM), jnp.float32),
            pltpu.VMEM((L, L), jnp.float32),
            pltpu.SMEM((NSLAB,), jnp.int32),
            pltpu.SMEM((NSLAB,), jnp.int32),
            pltpu.SemaphoreType.DMA,
            pltpu.SemaphoreType.DMA,
            pltpu.SemaphoreType.DMA,
            pltpu.SemaphoreType.DMA,
            pltpu.SemaphoreType.DMA,
            pltpu.SemaphoreType.DMA,
        ],
        interpret=interpret,
    )
    return kfn(trip, node_emb, w)


def kernel(triplets, node_emb, W):
    return _dist_mult_sc(triplets.astype(jnp.int32).T, node_emb, W)
